# baseline (speedup 1.0000x reference)
import functools

import jax
import jax.numpy as jnp
from jax import lax
from jax.experimental import pallas as pl
from jax.experimental.pallas import tpu as pltpu
from jax.experimental.pallas import tpu_sc as plsc

N = 10000
E = 320000
D = 128

NC = 2
NS = 16
NW = NC * NS

CH = 128
NCH = 80
NBUF = 2
NHALF = NCH // 2
DH = D // 2
EPW = NCH * CH
E_PAD = EPW * NW
NP = 10240
NPA = 10112

_mesh = plsc.VectorSubcoreMesh(
    core_axis_name="c", subcore_axis_name="s", num_cores=NC, num_subcores=NS)


_DEG_KW = dict(
    out_type=jax.ShapeDtypeStruct((NC, NP), jnp.float32),
    mesh=_mesh,
    scratch_types=[
        pltpu.VMEM((NCH, CH), jnp.int32),
        pltpu.VMEM((CH,), jnp.float32),
        pltpu.VMEM((16,), jnp.float32),
        pltpu.VMEM_SHARED((NP,), jnp.float32),
        [pltpu.SemaphoreType.DMA] * 4,
    ],
)


def _deg_body(dst_hbm, out_hbm, idx_v, ones_v, zero_v, deg_sh, sems):
    cid = lax.axis_index("c")
    sid = lax.axis_index("s")
    wid = cid * NS + sid

    for i in range(CH // 16):
        ones_v[pl.ds(i * 16, 16)] = jnp.full((16,), 1.0, jnp.float32)
    zero_v[...] = jnp.zeros((16,), jnp.float32)

    def zbody(k, carry):
        pltpu.sync_copy(zero_v, deg_sh.at[pl.ds(k * 16, 16)])
        return carry
    lax.fori_loop(sid * 40, sid * 40 + 40, zbody, 0)

    pltpu.sync_copy(dst_hbm.at[wid], idx_v)
    plsc.subcore_barrier()

    def body(k, carry):
        descs = [
            pltpu.async_copy(ones_v, deg_sh.at[idx_v.at[k * 4 + b]],
                             sems[b], add=True)
            for b in range(4)
        ]
        for d_ in descs:
            d_.wait()
        return carry
    lax.fori_loop(0, NCH // 4, body, 0)
    plsc.subcore_barrier()

    pltpu.sync_copy(deg_sh.at[pl.ds(sid * 640, 640)],
                    out_hbm.at[cid, pl.ds(sid * 640, 640)])


_deg_kernel = pl.kernel(_deg_body, **_DEG_KW)


_SCAT_KW = dict(
    out_type=jax.ShapeDtypeStruct((NC, NP, D), jnp.float32),
    mesh=_mesh,
    scratch_types=[
        pltpu.VMEM((CH,), jnp.int32),
        pltpu.VMEM((CH,), jnp.int32),
        pltpu.VMEM((CH, D), jnp.float32),
        pltpu.VMEM((16, D), jnp.float32),
        pltpu.VMEM_SHARED((NP, D), jnp.float32),
        pltpu.SemaphoreType.DMA,
    ],
)


def _scatter_body(src_hbm, dst_hbm, g_hbm, out_hbm,
                  src_v, dst_v, rows_v, zrow_v, acc_sh, sem):
    cid = lax.axis_index("c")
    sid = lax.axis_index("s")
    wid = cid * NS + sid

    for r in range(16):
        for j in range(D // 16):
            zrow_v[r, pl.ds(j * 16, 16)] = jnp.zeros((16,), jnp.float32)

    def zbody(k, carry):
        pltpu.sync_copy(zrow_v, acc_sh.at[pl.ds(k * 16, 16)])
        return carry
    lax.fori_loop(sid * 40, sid * 40 + 40, zbody, 0)
    plsc.subcore_barrier()

    def body(j, carry):
        base = wid * EPW + j * CH
        pltpu.sync_copy(src_hbm.at[pl.ds(base, CH)], src_v)
        pltpu.sync_copy(dst_hbm.at[pl.ds(base, CH)], dst_v)
        pltpu.async_copy(g_hbm.at[src_v], rows_v, sem).wait()
        pltpu.sync_copy(rows_v, acc_sh.at[dst_v], add=True)
        return carry
    lax.fori_loop(0, NCH, body, 0)
    plsc.subcore_barrier()

    pltpu.sync_copy(acc_sh.at[pl.ds(sid * 640, 640)],
                    out_hbm.at[cid, pl.ds(sid * 640, 640)])


_scatter_kernel = pl.kernel(_scatter_body, **_SCAT_KW)


_BLK = 1000


def _mm_body(x_ref, w_ref, o_ref):
    o_ref[...] = jnp.dot(x_ref[...], w_ref[...],
                         preferred_element_type=jnp.float32)


def _scale_body(h_ref, d_ref, o_ref):
    o_ref[...] = h_ref[...] * d_ref[...]


def _epi_body(a0_ref, a1_ref, g_ref, d_ref, b_ref, o_ref):
    acc = a0_ref[0] + a1_ref[0] + g_ref[...]
    o_ref[...] = d_ref[...] * acc + b_ref[0:1, :]


def kernel(x, edge_index, W, b):
    src = edge_index[0].astype(jnp.int32)
    dst = edge_index[1].astype(jnp.int32)
    pad = E_PAD - E
    src_p = jnp.concatenate([src, jnp.zeros((pad,), jnp.int32)])
    dst_p = jnp.concatenate([dst, jnp.full((pad,), N, jnp.int32)])
    dst_r = dst_p.reshape(NW, NCH, CH)

    h = pl.pallas_call(
        _mm_body,
        grid=(N // _BLK,),
        in_specs=[
            pl.BlockSpec((_BLK, D), lambda i: (i, 0)),
            pl.BlockSpec((D, D), lambda i: (0, 0)),
        ],
        out_specs=pl.BlockSpec((_BLK, D), lambda i: (i, 0)),
        out_shape=jax.ShapeDtypeStruct((N, D), jnp.float32),
    )(x, W)

    deg_part = _deg_kernel(dst_r)
    deg = (deg_part[0] + deg_part[1])[:N] + 1.0
    dis = lax.rsqrt(deg)
    dis_b = jnp.broadcast_to(dis[:, None], (N, D))

    g = pl.pallas_call(
        _scale_body,
        grid=(N // _BLK,),
        in_specs=[
            pl.BlockSpec((_BLK, D), lambda i: (i, 0)),
            pl.BlockSpec((_BLK, D), lambda i: (i, 0)),
        ],
        out_specs=pl.BlockSpec((_BLK, D), lambda i: (i, 0)),
        out_shape=jax.ShapeDtypeStruct((N, D), jnp.float32),
    )(h, dis_b)

    acc_part = _scatter_kernel(src_p, dst_p, g)

    b8 = jnp.broadcast_to(b[None, :], (8, D))
    out = pl.pallas_call(
        _epi_body,
        grid=(N // _BLK,),
        in_specs=[
            pl.BlockSpec((1, _BLK, D), lambda i: (0, i, 0)),
            pl.BlockSpec((1, _BLK, D), lambda i: (1, i, 0)),
            pl.BlockSpec((_BLK, D), lambda i: (i, 0)),
            pl.BlockSpec((_BLK, D), lambda i: (i, 0)),
            pl.BlockSpec((8, D), lambda i: (0, 0)),
        ],
        out_specs=pl.BlockSpec((_BLK, D), lambda i: (i, 0)),
        out_shape=jax.ShapeDtypeStruct((N, D), jnp.float32),
    )(acc_part[:, :N], acc_part[:, :N], g, dis_b, b8)
    return out


# speedup vs baseline: 2.2462x; 2.2462x over previous
import functools

import jax
import jax.numpy as jnp
from jax import lax
from jax.experimental import pallas as pl
from jax.experimental.pallas import tpu as pltpu
from jax.experimental.pallas import tpu_sc as plsc

N = 10000
E = 320000
D = 128

NC = 2
NS = 16
NW = NC * NS

CH = 128
NCH = 80
NBUF = 2
NHALF = NCH // 2
DH = D // 2
EPW = NCH * CH
E_PAD = EPW * NW
NP = 10240
NPA = 10112

_mesh = plsc.VectorSubcoreMesh(
    core_axis_name="c", subcore_axis_name="s", num_cores=NC, num_subcores=NS)


_DEG_KW = dict(
    out_type=jax.ShapeDtypeStruct((NC, NP), jnp.float32),
    mesh=_mesh,
    scratch_types=[
        pltpu.VMEM((NCH, CH), jnp.int32),
        pltpu.VMEM((CH,), jnp.float32),
        pltpu.VMEM((16,), jnp.float32),
        pltpu.VMEM_SHARED((NP,), jnp.float32),
        [pltpu.SemaphoreType.DMA] * 4,
    ],
)


def _deg_body(dst_hbm, out_hbm, idx_v, ones_v, zero_v, deg_sh, sems):
    cid = lax.axis_index("c")
    sid = lax.axis_index("s")
    wid = cid * NS + sid

    for i in range(CH // 16):
        ones_v[pl.ds(i * 16, 16)] = jnp.full((16,), 1.0, jnp.float32)
    zero_v[...] = jnp.zeros((16,), jnp.float32)

    def zbody(k, carry):
        pltpu.sync_copy(zero_v, deg_sh.at[pl.ds(k * 16, 16)])
        return carry
    lax.fori_loop(sid * 40, sid * 40 + 40, zbody, 0)

    pltpu.sync_copy(dst_hbm.at[wid], idx_v)
    plsc.subcore_barrier()

    def body(k, carry):
        descs = [
            pltpu.async_copy(ones_v, deg_sh.at[idx_v.at[k * 4 + b]],
                             sems[b], add=True)
            for b in range(4)
        ]
        for d_ in descs:
            d_.wait()
        return carry
    lax.fori_loop(0, NCH // 4, body, 0)
    plsc.subcore_barrier()

    pltpu.sync_copy(deg_sh.at[pl.ds(sid * 640, 640)],
                    out_hbm.at[cid, pl.ds(sid * 640, 640)])


_deg_kernel = pl.kernel(_deg_body, **_DEG_KW)


_SCAT_KW = dict(
    out_type=jax.ShapeDtypeStruct((NC, NP, D), jnp.float32),
    mesh=_mesh,
    scratch_types=[
        pltpu.VMEM((CH,), jnp.int32),
        pltpu.VMEM((CH,), jnp.int32),
        pltpu.VMEM((CH, D), jnp.float32),
        pltpu.VMEM((16, D), jnp.float32),
        pltpu.VMEM_SHARED((NP, D), jnp.float32),
        pltpu.SemaphoreType.DMA,
    ],
)


def _scatter_body(src_hbm, dst_hbm, g_hbm, out_hbm,
                  src_v, dst_v, rows_v, zrow_v, acc_sh, sem):
    cid = lax.axis_index("c")
    sid = lax.axis_index("s")
    wid = cid * NS + sid

    for r in range(16):
        for j in range(D // 16):
            zrow_v[r, pl.ds(j * 16, 16)] = jnp.zeros((16,), jnp.float32)

    def zbody(k, carry):
        pltpu.sync_copy(zrow_v, acc_sh.at[pl.ds(k * 16, 16)])
        return carry
    lax.fori_loop(sid * 40, sid * 40 + 40, zbody, 0)
    plsc.subcore_barrier()

    def body(j, carry):
        base = wid * EPW + j * CH
        pltpu.sync_copy(src_hbm.at[pl.ds(base, CH)], src_v)
        pltpu.sync_copy(dst_hbm.at[pl.ds(base, CH)], dst_v)
        pltpu.async_copy(g_hbm.at[src_v], rows_v, sem).wait()
        pltpu.sync_copy(rows_v, acc_sh.at[dst_v], add=True)
        return carry
    lax.fori_loop(0, NCH, body, 0)
    plsc.subcore_barrier()

    pltpu.sync_copy(acc_sh.at[pl.ds(sid * 640, 640)],
                    out_hbm.at[cid, pl.ds(sid * 640, 640)])


_scatter_kernel = pl.kernel(_scatter_body, **_SCAT_KW)


_BLK = 1000


def _mm_body(x_ref, w_ref, o_ref):
    o_ref[...] = jnp.dot(x_ref[...], w_ref[...],
                         preferred_element_type=jnp.float32)


def _scale_body(h_ref, d_ref, o_ref):
    o_ref[...] = h_ref[...] * d_ref[...]


def _epi_body(a0_ref, a1_ref, g_ref, d_ref, b_ref, o_ref):
    acc = a0_ref[0] + a1_ref[0] + g_ref[...]
    o_ref[...] = d_ref[...] * acc + b_ref[0:1, :]


def kernel(x, edge_index, W, b):
    src = edge_index[0].astype(jnp.int32)
    dst = edge_index[1].astype(jnp.int32)
    pad = E_PAD - E
    ar = jnp.arange(pad, dtype=jnp.int32)
    src_p = jnp.concatenate([src, (ar * 131) % N])
    dst_p = jnp.concatenate([dst, N + (ar % (NP - N))])
    dst_r = dst_p.reshape(NW, NCH, CH)

    h = pl.pallas_call(
        _mm_body,
        grid=(N // _BLK,),
        in_specs=[
            pl.BlockSpec((_BLK, D), lambda i: (i, 0)),
            pl.BlockSpec((D, D), lambda i: (0, 0)),
        ],
        out_specs=pl.BlockSpec((_BLK, D), lambda i: (i, 0)),
        out_shape=jax.ShapeDtypeStruct((N, D), jnp.float32),
    )(x, W)

    deg_part = _deg_kernel(dst_r)
    deg = (deg_part[0] + deg_part[1])[:N] + 1.0
    dis = lax.rsqrt(deg)
    dis_b = jnp.broadcast_to(dis[:, None], (N, D))

    g = pl.pallas_call(
        _scale_body,
        grid=(N // _BLK,),
        in_specs=[
            pl.BlockSpec((_BLK, D), lambda i: (i, 0)),
            pl.BlockSpec((_BLK, D), lambda i: (i, 0)),
        ],
        out_specs=pl.BlockSpec((_BLK, D), lambda i: (i, 0)),
        out_shape=jax.ShapeDtypeStruct((N, D), jnp.float32),
    )(h, dis_b)

    acc_part = _scatter_kernel(src_p, dst_p, g)

    b8 = jnp.broadcast_to(b[None, :], (8, D))
    out = pl.pallas_call(
        _epi_body,
        grid=(N // _BLK,),
        in_specs=[
            pl.BlockSpec((1, _BLK, D), lambda i: (0, i, 0)),
            pl.BlockSpec((1, _BLK, D), lambda i: (1, i, 0)),
            pl.BlockSpec((_BLK, D), lambda i: (i, 0)),
            pl.BlockSpec((_BLK, D), lambda i: (i, 0)),
            pl.BlockSpec((8, D), lambda i: (0, 0)),
        ],
        out_specs=pl.BlockSpec((_BLK, D), lambda i: (i, 0)),
        out_shape=jax.ShapeDtypeStruct((N, D), jnp.float32),
    )(acc_part[:, :N], acc_part[:, :N], g, dis_b, b8)
    return out
